# baseline (speedup 1.0000x reference)
import functools

import jax
import jax.numpy as jnp
from jax import lax
from jax.experimental import pallas as pl
from jax.experimental.pallas import tpu as pltpu
from jax.experimental.pallas import tpu_sc as plsc

N = 10000
E = 320000
D = 128
H = 128
NE = 3

NC = 2
NS = 16
L = 16
NT = NC * NS
EPT = E // NT
CH = 80

SB = 2000
NSB = EPT // SB
CPS = SB // CH

NP = 16384
CROWS = NE * NP
CRPT = CROWS // NS

_mesh = lambda: plsc.VectorSubcoreMesh(core_axis_name="c", subcore_axis_name="s")
_sc_params = pltpu.CompilerParams(needs_layout_passes=False)


@functools.partial(
    pl.kernel,
    out_type=jax.ShapeDtypeStruct((NC * CROWS,), jnp.float32),
    mesh=_mesh(),
    compiler_params=_sc_params,
    scratch_types=[
        pltpu.VMEM_SHARED((CROWS,), jnp.float32),
        pltpu.VMEM((SB,), jnp.int32),
        pltpu.VMEM((CH,), jnp.int32),
        pltpu.VMEM((CH,), jnp.int32),
        pltpu.VMEM((CH,), jnp.float32),
        pltpu.VMEM((CRPT,), jnp.float32),
        pltpu.SemaphoreType.DMA,
        pltpu.SemaphoreType.DMA,
    ],
)
def _sc_counts(cidx_hbm, out_hbm, cnt_sh, csb_v, cc0_v, cc1_v, ones_v, z_v,
               sem0, sem1):
    c = lax.axis_index("c")
    s = lax.axis_index("s")
    wid = c * NS + s

    zero16 = jnp.zeros((L,), jnp.float32)
    one16 = jnp.ones((L,), jnp.float32)

    for q in range(CH // L):
        ones_v[pl.ds(q * L, L)] = one16

    def fill_z(i, carry):
        z_v[pl.ds(i * L, L)] = zero16
        return carry
    lax.fori_loop(0, CRPT // L, fill_z, 0)

    pltpu.sync_copy(z_v, cnt_sh.at[pl.ds(s * CRPT, CRPT)])
    plsc.subcore_barrier()

    base = wid * EPT
    bufs = ((cc0_v, sem0), (cc1_v, sem1))

    def stage(k, b):
        cc_v, _ = bufs[b]
        for q in range(CH // L):
            cc_v[pl.ds(q * L, L)] = csb_v[pl.ds(k * CH + q * L, L)]

    def fire(b):
        cc_v, sem = bufs[b]
        pltpu.async_copy(ones_v, cnt_sh.at[cc_v], sem, add=True)

    def drain(b):
        cc_v, sem = bufs[b]
        pltpu.make_async_copy(ones_v, cnt_sh.at[cc_v], sem).wait()

    for sb in range(NSB):
        off = pl.multiple_of(base + sb * SB, 8)
        pltpu.sync_copy(cidx_hbm.at[pl.ds(off, SB)], csb_v)
        stage(0, 0)
        fire(0)
        stage(1, 1)
        fire(1)

        def chunk2(i, carry):
            drain(0)
            stage(2 * i + 2, 0)
            fire(0)
            drain(1)

            @pl.when(2 * i + 3 < CPS)
            def _():
                stage(2 * i + 3, 1)
                fire(1)
            return carry
        lax.fori_loop(0, (CPS - 2) // 2, chunk2, 0)
        drain(0)
        drain(1)
        stage(CPS - 1, 0)
        fire(0)
        drain(0)
    plsc.subcore_barrier()

    pltpu.sync_copy(cnt_sh.at[pl.ds(s * CRPT, CRPT)],
                    out_hbm.at[pl.ds(c * CROWS + s * CRPT, CRPT)])


NBUF = 4


@functools.partial(
    pl.kernel,
    out_type=jax.ShapeDtypeStruct((NC, N, H), jnp.float32),
    mesh=_mesh(),
    compiler_params=_sc_params,
    scratch_types=[
        pltpu.VMEM_SHARED((N, H), jnp.float32),
        pltpu.VMEM((SB,), jnp.int32),
        pltpu.VMEM((SB,), jnp.int32),
        [pltpu.VMEM((CH,), jnp.int32) for _ in range(NBUF)],
        [pltpu.VMEM((CH,), jnp.float32) for _ in range(NBUF)],
        [pltpu.VMEM((CH, H), jnp.float32) for _ in range(NBUF)],
        [pltpu.SemaphoreType.DMA for _ in range(NBUF)],
        [pltpu.SemaphoreType.DMA for _ in range(NBUF)],
        [pltpu.SemaphoreType.DMA for _ in range(NBUF)],
    ],
)
def _sc_scatter(y_hbm, gidx_hbm, cidx_hbm, cinv_hbm, out_hbm,
                agg_sh, gsb_v, csb_v, dc_vs, w_vs, rows_vs,
                gsems, wsems, ssems):
    c = lax.axis_index("c")
    s = lax.axis_index("s")
    wid = c * NS + s

    zero16 = jnp.zeros((L,), jnp.float32)
    rows0_v = rows_vs[0]

    def fill_z(i, carry):
        for j in range(H // L):
            rows0_v[i, pl.ds(j * L, L)] = zero16
        return carry
    lax.fori_loop(0, CH, fill_z, 0)

    zstart = pl.multiple_of(s * 624, 8)
    for i in range(7):
        pltpu.sync_copy(rows0_v, agg_sh.at[pl.ds(zstart + i * CH, CH)])
    pltpu.sync_copy(rows0_v.at[pl.ds(0, 64)],
                    agg_sh.at[pl.ds(zstart + 7 * CH, 64)])

    @pl.when(s == NS - 1)
    def _():
        pltpu.sync_copy(rows0_v.at[pl.ds(0, 16)], agg_sh.at[pl.ds(N - 16, 16)])

    plsc.subcore_barrier()

    base = wid * EPT

    def pre(k, b, wait_scatter):
        if wait_scatter:
            pltpu.make_async_copy(rows_vs[b], agg_sh.at[dc_vs[b]],
                                  ssems[b]).wait()
        for q in range(CH // L):
            ci = csb_v[pl.ds(k * CH + q * L, L)]
            dc_vs[b][pl.ds(q * L, L)] = ci & (NP - 1)
        gsl = gsb_v.at[pl.ds(k * CH, CH)]
        csl = csb_v.at[pl.ds(k * CH, CH)]
        pltpu.async_copy(y_hbm.at[gsl], rows_vs[b], gsems[b])
        pltpu.async_copy(cinv_hbm.at[csl], w_vs[b], wsems[b])

    def mid(k, b):
        gsl = gsb_v.at[pl.ds(k * CH, CH)]
        csl = csb_v.at[pl.ds(k * CH, CH)]
        pltpu.make_async_copy(y_hbm.at[gsl], rows_vs[b], gsems[b]).wait()
        pltpu.make_async_copy(cinv_hbm.at[csl], w_vs[b], wsems[b]).wait()
        w_v = w_vs[b]
        rows_v = rows_vs[b]

        def scale(r, carry2):
            r2 = r * 2
            wr = plsc.load_gather(w_v, [jnp.full((L,), r2, jnp.int32)])
            wr2 = plsc.load_gather(w_v, [jnp.full((L,), r2 + 1, jnp.int32)])
            for j in range(H // L):
                sl = pl.ds(j * L, L)
                rows_v[r2, sl] = rows_v[r2, sl] * wr
                rows_v[r2 + 1, sl] = rows_v[r2 + 1, sl] * wr2
            return carry2
        lax.fori_loop(0, CH // 2, scale, 0)
        pltpu.async_copy(rows_v, agg_sh.at[dc_vs[b]], ssems[b], add=True)

    for sb in range(NSB):
        off = pl.multiple_of(base + sb * SB, 8)
        pltpu.sync_copy(gidx_hbm.at[pl.ds(off, SB)], gsb_v)
        pltpu.sync_copy(cidx_hbm.at[pl.ds(off, SB)], csb_v)
        for b in range(NBUF):
            pre(b, b, wait_scatter=(sb > 0))

        def ring(i, carry):
            k = i * NBUF
            for b in range(NBUF):
                mid(k + b, b)
            for b in range(NBUF):
                pre(k + NBUF + b, b, wait_scatter=True)
            return carry
        lax.fori_loop(0, CPS // NBUF - 1, ring, 0)
        k0 = CPS - CPS % NBUF - NBUF
        for b in range(NBUF):
            mid(k0 + b, b)
        for k in range(CPS - CPS % NBUF, CPS):
            pre(k, k % NBUF, wait_scatter=True)
            mid(k, k % NBUF)
    for b in range(NBUF):
        pltpu.make_async_copy(rows_vs[b], agg_sh.at[dc_vs[b]], ssems[b]).wait()
    plsc.subcore_barrier()

    pltpu.sync_copy(agg_sh.at[pl.ds(zstart, 624)],
                    out_hbm.at[c, pl.ds(zstart, 624)])

    @pl.when(s == NS - 1)
    def _():
        pltpu.sync_copy(agg_sh.at[pl.ds(N - 16, 16)],
                        out_hbm.at[c, pl.ds(N - 16, 16)])


_PREC = lax.Precision.HIGHEST
BN = 1000
NB = N // BN
ER = E // 128


def _tc_prep_body(src_ref, dst_ref, et_ref, gidx_ref, cidx_ref):
    et = et_ref[...]
    gidx_ref[...] = et * N + src_ref[...]
    cidx_ref[...] = et * NP + dst_ref[...]


def _tc_prep(src, dst, et):
    gidx, cidx = pl.pallas_call(
        _tc_prep_body,
        out_shape=[
            jax.ShapeDtypeStruct((ER, 128), jnp.int32),
            jax.ShapeDtypeStruct((ER, 128), jnp.int32),
        ],
    )(src.reshape(ER, 128), dst.reshape(ER, 128), et.reshape(ER, 128))
    return gidx.reshape(E), cidx.reshape(E)


def _tc_cinv_body(cnt_ref, cinv_ref):
    cnt = cnt_ref[0] + cnt_ref[1]
    cinv_ref[...] = 1.0 / (3.0 * jnp.maximum(cnt, 1.0))


def _tc_cinv(cnt_part):
    c2 = cnt_part.reshape(NC, CROWS // 128, 128)
    out = pl.pallas_call(
        _tc_cinv_body,
        out_shape=jax.ShapeDtypeStruct((CROWS // 128, 128), jnp.float32),
    )(c2)
    return out.reshape(CROWS)


def _tc_dense1_body(x_ref, wn_ref, wr_ref, b_ref, y_ref, xr_ref):
    xb = x_ref[...]
    for e in range(NE):
        y_ref[e] = jnp.dot(xb, wn_ref[e], precision=_PREC)
    wrm = (wr_ref[0] + wr_ref[1] + wr_ref[2]) * (1.0 / 3.0)
    bm = (b_ref[0] + b_ref[1] + b_ref[2]) * (1.0 / 3.0)
    xr_ref[...] = jnp.dot(xb, wrm, precision=_PREC) + bm[None, :]


def _tc_dense1(x, Wn, Wr, b):
    return pl.pallas_call(
        _tc_dense1_body,
        grid=(NB,),
        in_specs=[
            pl.BlockSpec((BN, D), lambda i: (i, 0)),
            pl.BlockSpec((NE, D, H), lambda i: (0, 0, 0)),
            pl.BlockSpec((NE, D, H), lambda i: (0, 0, 0)),
            pl.BlockSpec((NE, H), lambda i: (0, 0)),
        ],
        out_specs=[
            pl.BlockSpec((NE, BN, H), lambda i: (0, i, 0)),
            pl.BlockSpec((BN, H), lambda i: (i, 0)),
        ],
        out_shape=[
            jax.ShapeDtypeStruct((NE, N, H), jnp.float32),
            jax.ShapeDtypeStruct((N, H), jnp.float32),
        ],
    )(x, Wn, Wr, b)


def _tc_dense2_body(xr_ref, agg_ref, wn_ref, wr_ref, b_ref, y_ref, xr2_ref):
    h = xr_ref[...] + agg_ref[0] + agg_ref[1]
    nrm = jnp.sqrt(jnp.sum(h * h, axis=-1, keepdims=True))
    h = h / jnp.maximum(nrm, 1e-12)
    h = jnp.maximum(h, 0.0)
    for e in range(NE):
        y_ref[e] = jnp.dot(h, wn_ref[e], precision=_PREC)
    wrm = (wr_ref[0] + wr_ref[1] + wr_ref[2]) * (1.0 / 3.0)
    bm = (b_ref[0] + b_ref[1] + b_ref[2]) * (1.0 / 3.0)
    xr2_ref[...] = jnp.dot(h, wrm, precision=_PREC) + bm[None, :]


def _tc_dense2(xr, agg, Wn, Wr, b):
    return pl.pallas_call(
        _tc_dense2_body,
        grid=(NB,),
        in_specs=[
            pl.BlockSpec((BN, H), lambda i: (i, 0)),
            pl.BlockSpec((NC, BN, H), lambda i: (0, i, 0)),
            pl.BlockSpec((NE, H, H), lambda i: (0, 0, 0)),
            pl.BlockSpec((NE, H, H), lambda i: (0, 0, 0)),
            pl.BlockSpec((NE, H), lambda i: (0, 0)),
        ],
        out_specs=[
            pl.BlockSpec((NE, BN, H), lambda i: (0, i, 0)),
            pl.BlockSpec((BN, H), lambda i: (i, 0)),
        ],
        out_shape=[
            jax.ShapeDtypeStruct((NE, N, H), jnp.float32),
            jax.ShapeDtypeStruct((N, H), jnp.float32),
        ],
    )(xr, agg, Wn, Wr, b)


def _tc_final_body(xr_ref, agg_ref, out_ref):
    out_ref[...] = xr_ref[...] + agg_ref[0] + agg_ref[1]


def _tc_final(xr, agg):
    return pl.pallas_call(
        _tc_final_body,
        grid=(NB,),
        in_specs=[
            pl.BlockSpec((BN, H), lambda i: (i, 0)),
            pl.BlockSpec((NC, BN, H), lambda i: (0, i, 0)),
        ],
        out_specs=pl.BlockSpec((BN, H), lambda i: (i, 0)),
        out_shape=jax.ShapeDtypeStruct((N, H), jnp.float32),
    )(xr, agg)


def kernel(x, edge_index, edge_type, beat_nodes, measure_nodes, beat_edges,
           measure_edges, W1r, W1n, b1, W2r, W2n, b2):
    del beat_nodes, measure_nodes, beat_edges, measure_edges
    src = edge_index[0]
    dst = edge_index[1]

    gidx, cidx = _tc_prep(src, dst, edge_type)
    cnt_part = _sc_counts(cidx)
    cinv = _tc_cinv(cnt_part)

    y1, xr1 = _tc_dense1(x, W1n, W1r, b1)
    agg1 = _sc_scatter(y1.reshape(NE * N, H), gidx, cidx, cinv)

    y2, xr2 = _tc_dense2(xr1, agg1, W2n, W2r, b2)
    agg2 = _sc_scatter(y2.reshape(NE * N, H), gidx, cidx, cinv)

    return _tc_final(xr2, agg2)


# speedup vs baseline: 1.0335x; 1.0335x over previous
import functools

import jax
import jax.numpy as jnp
from jax import lax
from jax.experimental import pallas as pl
from jax.experimental.pallas import tpu as pltpu
from jax.experimental.pallas import tpu_sc as plsc

N = 10000
E = 320000
D = 128
H = 128
NE = 3

NC = 2
NS = 16
L = 16
NT = NC * NS
EPT = E // NT
CH = 80

SB = 2000
NSB = EPT // SB
CPS = SB // CH

NP = 16384
CROWS = NE * NP
CRPT = CROWS // NS

_mesh = lambda: plsc.VectorSubcoreMesh(core_axis_name="c", subcore_axis_name="s")
_sc_params = pltpu.CompilerParams(needs_layout_passes=False)


@functools.partial(
    pl.kernel,
    out_type=jax.ShapeDtypeStruct((NC * CROWS,), jnp.float32),
    mesh=_mesh(),
    compiler_params=_sc_params,
    scratch_types=[
        pltpu.VMEM_SHARED((CROWS,), jnp.float32),
        pltpu.VMEM((SB,), jnp.int32),
        pltpu.VMEM((SB,), jnp.int32),
        pltpu.VMEM((CH,), jnp.int32),
        pltpu.VMEM((CH,), jnp.int32),
        pltpu.VMEM((CH,), jnp.float32),
        pltpu.VMEM((CRPT,), jnp.float32),
        pltpu.SemaphoreType.DMA,
        pltpu.SemaphoreType.DMA,
    ],
)
def _sc_counts(et_hbm, dst_hbm, out_hbm, cnt_sh, esb_v, dsb_v, cc0_v, cc1_v,
               ones_v, z_v, sem0, sem1):
    c = lax.axis_index("c")
    s = lax.axis_index("s")
    wid = c * NS + s

    zero16 = jnp.zeros((L,), jnp.float32)
    one16 = jnp.ones((L,), jnp.float32)

    for q in range(CH // L):
        ones_v[pl.ds(q * L, L)] = one16

    def fill_z(i, carry):
        z_v[pl.ds(i * L, L)] = zero16
        return carry
    lax.fori_loop(0, CRPT // L, fill_z, 0)

    pltpu.sync_copy(z_v, cnt_sh.at[pl.ds(s * CRPT, CRPT)])
    plsc.subcore_barrier()

    base = wid * EPT
    bufs = ((cc0_v, sem0), (cc1_v, sem1))

    def stage(k, b):
        cc_v, _ = bufs[b]
        for q in range(CH // L):
            ev = esb_v[pl.ds(k * CH + q * L, L)]
            dv = dsb_v[pl.ds(k * CH + q * L, L)]
            cc_v[pl.ds(q * L, L)] = ev * NP + dv

    def fire(b):
        cc_v, sem = bufs[b]
        pltpu.async_copy(ones_v, cnt_sh.at[cc_v], sem, add=True)

    def drain(b):
        cc_v, sem = bufs[b]
        pltpu.make_async_copy(ones_v, cnt_sh.at[cc_v], sem).wait()

    for sb in range(NSB):
        off = pl.multiple_of(base + sb * SB, 8)
        pltpu.sync_copy(et_hbm.at[pl.ds(off, SB)], esb_v)
        pltpu.sync_copy(dst_hbm.at[pl.ds(off, SB)], dsb_v)
        stage(0, 0)
        fire(0)
        stage(1, 1)
        fire(1)

        def chunk2(i, carry):
            drain(0)
            stage(2 * i + 2, 0)
            fire(0)
            drain(1)

            @pl.when(2 * i + 3 < CPS)
            def _():
                stage(2 * i + 3, 1)
                fire(1)
            return carry
        lax.fori_loop(0, (CPS - 2) // 2, chunk2, 0)
        drain(0)
        drain(1)
        stage(CPS - 1, 0)
        fire(0)
        drain(0)
    plsc.subcore_barrier()

    pltpu.sync_copy(cnt_sh.at[pl.ds(s * CRPT, CRPT)],
                    out_hbm.at[pl.ds(c * CROWS + s * CRPT, CRPT)])


NBUF = 4


@functools.partial(
    pl.kernel,
    out_type=jax.ShapeDtypeStruct((NC, N, H), jnp.float32),
    mesh=_mesh(),
    compiler_params=_sc_params,
    scratch_types=[
        pltpu.VMEM_SHARED((N, H), jnp.float32),
        pltpu.VMEM((SB,), jnp.int32),
        pltpu.VMEM((SB,), jnp.int32),
        [pltpu.VMEM((CH,), jnp.int32) for _ in range(NBUF)],
        [pltpu.VMEM((CH,), jnp.float32) for _ in range(NBUF)],
        [pltpu.VMEM((CH, H), jnp.float32) for _ in range(NBUF)],
        [pltpu.SemaphoreType.DMA for _ in range(NBUF)],
        [pltpu.SemaphoreType.DMA for _ in range(NBUF)],
        [pltpu.SemaphoreType.DMA for _ in range(NBUF)],
    ],
)
def _sc_scatter(y_hbm, gidx_hbm, cidx_hbm, cinv_hbm, out_hbm,
                agg_sh, gsb_v, csb_v, dc_vs, w_vs, rows_vs,
                gsems, wsems, ssems):
    c = lax.axis_index("c")
    s = lax.axis_index("s")
    wid = c * NS + s

    zero16 = jnp.zeros((L,), jnp.float32)
    rows0_v = rows_vs[0]

    def fill_z(i, carry):
        for j in range(H // L):
            rows0_v[i, pl.ds(j * L, L)] = zero16
        return carry
    lax.fori_loop(0, CH, fill_z, 0)

    zstart = pl.multiple_of(s * 624, 8)
    for i in range(7):
        pltpu.sync_copy(rows0_v, agg_sh.at[pl.ds(zstart + i * CH, CH)])
    pltpu.sync_copy(rows0_v.at[pl.ds(0, 64)],
                    agg_sh.at[pl.ds(zstart + 7 * CH, 64)])

    @pl.when(s == NS - 1)
    def _():
        pltpu.sync_copy(rows0_v.at[pl.ds(0, 16)], agg_sh.at[pl.ds(N - 16, 16)])

    plsc.subcore_barrier()

    base = wid * EPT

    def pre(k, b, wait_scatter):
        if wait_scatter:
            pltpu.make_async_copy(rows_vs[b], agg_sh.at[dc_vs[b]],
                                  ssems[b]).wait()
        for q in range(CH // L):
            ci = csb_v[pl.ds(k * CH + q * L, L)]
            dc_vs[b][pl.ds(q * L, L)] = ci & (NP - 1)
        gsl = gsb_v.at[pl.ds(k * CH, CH)]
        csl = csb_v.at[pl.ds(k * CH, CH)]
        pltpu.async_copy(y_hbm.at[gsl], rows_vs[b], gsems[b])
        pltpu.async_copy(cinv_hbm.at[csl], w_vs[b], wsems[b])

    def mid(k, b):
        gsl = gsb_v.at[pl.ds(k * CH, CH)]
        csl = csb_v.at[pl.ds(k * CH, CH)]
        pltpu.make_async_copy(y_hbm.at[gsl], rows_vs[b], gsems[b]).wait()
        pltpu.make_async_copy(cinv_hbm.at[csl], w_vs[b], wsems[b]).wait()
        w_v = w_vs[b]
        rows_v = rows_vs[b]

        def scale(r, carry2):
            r4 = r * 4
            ws = [plsc.load_gather(w_v, [jnp.full((L,), r4 + t, jnp.int32)])
                  for t in range(4)]
            for j in range(H // L):
                sl = pl.ds(j * L, L)
                for t in range(4):
                    rows_v[r4 + t, sl] = rows_v[r4 + t, sl] * ws[t]
            return carry2
        lax.fori_loop(0, CH // 4, scale, 0)
        pltpu.async_copy(rows_v, agg_sh.at[dc_vs[b]], ssems[b], add=True)

    for sb in range(NSB):
        off = pl.multiple_of(base + sb * SB, 8)
        pltpu.sync_copy(gidx_hbm.at[pl.ds(off, SB)], gsb_v)
        pltpu.sync_copy(cidx_hbm.at[pl.ds(off, SB)], csb_v)
        for b in range(NBUF):
            pre(b, b, wait_scatter=(sb > 0))

        def ring(i, carry):
            k = i * NBUF
            for b in range(NBUF):
                mid(k + b, b)
            for b in range(NBUF):
                pre(k + NBUF + b, b, wait_scatter=True)
            return carry
        lax.fori_loop(0, CPS // NBUF - 1, ring, 0)
        k0 = CPS - CPS % NBUF - NBUF
        for b in range(NBUF):
            mid(k0 + b, b)
        for k in range(CPS - CPS % NBUF, CPS):
            pre(k, k % NBUF, wait_scatter=True)
            mid(k, k % NBUF)
    for b in range(NBUF):
        pltpu.make_async_copy(rows_vs[b], agg_sh.at[dc_vs[b]], ssems[b]).wait()
    plsc.subcore_barrier()

    pltpu.sync_copy(agg_sh.at[pl.ds(zstart, 624)],
                    out_hbm.at[c, pl.ds(zstart, 624)])

    @pl.when(s == NS - 1)
    def _():
        pltpu.sync_copy(agg_sh.at[pl.ds(N - 16, 16)],
                        out_hbm.at[c, pl.ds(N - 16, 16)])


_PREC = lax.Precision.HIGHEST
BN = 1000
NB = N // BN
ER = E // 128


def _tc_cinv_body(cnt_ref, cinv_ref):
    cnt = cnt_ref[0] + cnt_ref[1]
    cinv_ref[...] = 1.0 / (3.0 * jnp.maximum(cnt, 1.0))


def _tc_cinv(cnt_part):
    c2 = cnt_part.reshape(NC, CROWS // 128, 128)
    out = pl.pallas_call(
        _tc_cinv_body,
        out_shape=jax.ShapeDtypeStruct((CROWS // 128, 128), jnp.float32),
    )(c2)
    return out.reshape(CROWS)


def _tc_prep_body(src_ref, dst_ref, et_ref, gidx_ref, cidx_ref):
    et = et_ref[...]
    gidx_ref[...] = et * N + src_ref[...]
    cidx_ref[...] = et * NP + dst_ref[...]


def _tc_prep(src, dst, et):
    gidx, cidx = pl.pallas_call(
        _tc_prep_body,
        out_shape=[
            jax.ShapeDtypeStruct((ER, 128), jnp.int32),
            jax.ShapeDtypeStruct((ER, 128), jnp.int32),
        ],
    )(src.reshape(ER, 128), dst.reshape(ER, 128), et.reshape(ER, 128))
    return gidx.reshape(E), cidx.reshape(E)


def _tc_dense1_body(x_ref, wn_ref, wr_ref, b_ref, y_ref, xr_ref):
    xb = x_ref[...]
    for e in range(NE):
        y_ref[e] = jnp.dot(xb, wn_ref[e], precision=_PREC)
    wrm = (wr_ref[0] + wr_ref[1] + wr_ref[2]) * (1.0 / 3.0)
    bm = (b_ref[0] + b_ref[1] + b_ref[2]) * (1.0 / 3.0)
    xr_ref[...] = jnp.dot(xb, wrm, precision=_PREC) + bm[None, :]


def _tc_dense1(x, Wn, Wr, b):
    return pl.pallas_call(
        _tc_dense1_body,
        grid=(NB,),
        in_specs=[
            pl.BlockSpec((BN, D), lambda i: (i, 0)),
            pl.BlockSpec((NE, D, H), lambda i: (0, 0, 0)),
            pl.BlockSpec((NE, D, H), lambda i: (0, 0, 0)),
            pl.BlockSpec((NE, H), lambda i: (0, 0)),
        ],
        out_specs=[
            pl.BlockSpec((NE, BN, H), lambda i: (0, i, 0)),
            pl.BlockSpec((BN, H), lambda i: (i, 0)),
        ],
        out_shape=[
            jax.ShapeDtypeStruct((NE, N, H), jnp.float32),
            jax.ShapeDtypeStruct((N, H), jnp.float32),
        ],
    )(x, Wn, Wr, b)


def _tc_dense2_body(xr_ref, agg_ref, wn_ref, wr_ref, b_ref, y_ref, xr2_ref):
    h = xr_ref[...] + agg_ref[0] + agg_ref[1]
    nrm = jnp.sqrt(jnp.sum(h * h, axis=-1, keepdims=True))
    h = h / jnp.maximum(nrm, 1e-12)
    h = jnp.maximum(h, 0.0)
    for e in range(NE):
        y_ref[e] = jnp.dot(h, wn_ref[e], precision=_PREC)
    wrm = (wr_ref[0] + wr_ref[1] + wr_ref[2]) * (1.0 / 3.0)
    bm = (b_ref[0] + b_ref[1] + b_ref[2]) * (1.0 / 3.0)
    xr2_ref[...] = jnp.dot(h, wrm, precision=_PREC) + bm[None, :]


def _tc_dense2(xr, agg, Wn, Wr, b):
    return pl.pallas_call(
        _tc_dense2_body,
        grid=(NB,),
        in_specs=[
            pl.BlockSpec((BN, H), lambda i: (i, 0)),
            pl.BlockSpec((NC, BN, H), lambda i: (0, i, 0)),
            pl.BlockSpec((NE, H, H), lambda i: (0, 0, 0)),
            pl.BlockSpec((NE, H, H), lambda i: (0, 0, 0)),
            pl.BlockSpec((NE, H), lambda i: (0, 0)),
        ],
        out_specs=[
            pl.BlockSpec((NE, BN, H), lambda i: (0, i, 0)),
            pl.BlockSpec((BN, H), lambda i: (i, 0)),
        ],
        out_shape=[
            jax.ShapeDtypeStruct((NE, N, H), jnp.float32),
            jax.ShapeDtypeStruct((N, H), jnp.float32),
        ],
    )(xr, agg, Wn, Wr, b)


def _tc_final_body(xr_ref, agg_ref, out_ref):
    out_ref[...] = xr_ref[...] + agg_ref[0] + agg_ref[1]


def _tc_final(xr, agg):
    return pl.pallas_call(
        _tc_final_body,
        grid=(NB,),
        in_specs=[
            pl.BlockSpec((BN, H), lambda i: (i, 0)),
            pl.BlockSpec((NC, BN, H), lambda i: (0, i, 0)),
        ],
        out_specs=pl.BlockSpec((BN, H), lambda i: (i, 0)),
        out_shape=jax.ShapeDtypeStruct((N, H), jnp.float32),
    )(xr, agg)


def kernel(x, edge_index, edge_type, beat_nodes, measure_nodes, beat_edges,
           measure_edges, W1r, W1n, b1, W2r, W2n, b2):
    del beat_nodes, measure_nodes, beat_edges, measure_edges
    src = edge_index[0]
    dst = edge_index[1]

    cnt_part = _sc_counts(edge_type, dst)
    cinv = _tc_cinv(cnt_part)

    gidx, cidx = _tc_prep(src, dst, edge_type)
    y1, xr1 = _tc_dense1(x, W1n, W1r, b1)
    agg1 = _sc_scatter(y1.reshape(NE * N, H), gidx, cidx, cinv)

    y2, xr2 = _tc_dense2(xr1, agg1, W2n, W2r, b2)
    agg2 = _sc_scatter(y2.reshape(NE * N, H), gidx, cidx, cinv)

    return _tc_final(xr2, agg2)


# speedup vs baseline: 1.0343x; 1.0009x over previous
import functools

import jax
import jax.numpy as jnp
from jax import lax
from jax.experimental import pallas as pl
from jax.experimental.pallas import tpu as pltpu
from jax.experimental.pallas import tpu_sc as plsc

N = 10000
E = 320000
D = 128
H = 128
NE = 3

NC = 2
NS = 16
L = 16
NT = NC * NS
EPT = E // NT
CH = 80

SB = 2000
NSB = EPT // SB
CPS = SB // CH

NP = 16384
CROWS = NE * NP
CRPT = CROWS // NS

_mesh = lambda: plsc.VectorSubcoreMesh(core_axis_name="c", subcore_axis_name="s")
_sc_params = pltpu.CompilerParams(needs_layout_passes=False)


@functools.partial(
    pl.kernel,
    out_type=jax.ShapeDtypeStruct((NC * CROWS,), jnp.float32),
    mesh=_mesh(),
    compiler_params=_sc_params,
    scratch_types=[
        pltpu.VMEM_SHARED((CROWS,), jnp.float32),
        pltpu.VMEM((SB,), jnp.int32),
        pltpu.VMEM((SB,), jnp.int32),
        pltpu.VMEM((CH,), jnp.int32),
        pltpu.VMEM((CH,), jnp.int32),
        pltpu.VMEM((CH,), jnp.float32),
        pltpu.VMEM((CRPT,), jnp.float32),
        pltpu.SemaphoreType.DMA,
        pltpu.SemaphoreType.DMA,
    ],
)
def _sc_counts(et_hbm, dst_hbm, out_hbm, cnt_sh, esb_v, dsb_v, cc0_v, cc1_v,
               ones_v, z_v, sem0, sem1):
    c = lax.axis_index("c")
    s = lax.axis_index("s")
    wid = c * NS + s

    zero16 = jnp.zeros((L,), jnp.float32)
    one16 = jnp.ones((L,), jnp.float32)

    for q in range(CH // L):
        ones_v[pl.ds(q * L, L)] = one16

    def fill_z(i, carry):
        z_v[pl.ds(i * L, L)] = zero16
        return carry
    lax.fori_loop(0, CRPT // L, fill_z, 0)

    pltpu.sync_copy(z_v, cnt_sh.at[pl.ds(s * CRPT, CRPT)])
    plsc.subcore_barrier()

    base = wid * EPT
    bufs = ((cc0_v, sem0), (cc1_v, sem1))

    def stage(k, b):
        cc_v, _ = bufs[b]
        for q in range(CH // L):
            ev = esb_v[pl.ds(k * CH + q * L, L)]
            dv = dsb_v[pl.ds(k * CH + q * L, L)]
            cc_v[pl.ds(q * L, L)] = ev * NP + dv

    def fire(b):
        cc_v, sem = bufs[b]
        pltpu.async_copy(ones_v, cnt_sh.at[cc_v], sem, add=True)

    def drain(b):
        cc_v, sem = bufs[b]
        pltpu.make_async_copy(ones_v, cnt_sh.at[cc_v], sem).wait()

    for sb in range(NSB):
        off = pl.multiple_of(base + sb * SB, 8)
        pltpu.sync_copy(et_hbm.at[pl.ds(off, SB)], esb_v)
        pltpu.sync_copy(dst_hbm.at[pl.ds(off, SB)], dsb_v)
        stage(0, 0)
        fire(0)
        stage(1, 1)
        fire(1)

        def chunk2(i, carry):
            drain(0)
            stage(2 * i + 2, 0)
            fire(0)
            drain(1)

            @pl.when(2 * i + 3 < CPS)
            def _():
                stage(2 * i + 3, 1)
                fire(1)
            return carry
        lax.fori_loop(0, (CPS - 2) // 2, chunk2, 0)
        drain(0)
        drain(1)
        if CPS % 2 == 1:
            stage(CPS - 1, 0)
            fire(0)
            drain(0)
    plsc.subcore_barrier()

    pltpu.sync_copy(cnt_sh.at[pl.ds(s * CRPT, CRPT)],
                    out_hbm.at[pl.ds(c * CROWS + s * CRPT, CRPT)])


NBUF = 4


@functools.partial(
    pl.kernel,
    out_type=jax.ShapeDtypeStruct((NC, N, H), jnp.float32),
    mesh=_mesh(),
    compiler_params=_sc_params,
    scratch_types=[
        pltpu.VMEM_SHARED((N, H), jnp.float32),
        pltpu.VMEM((SB,), jnp.int32),
        pltpu.VMEM((SB,), jnp.int32),
        [pltpu.VMEM((CH,), jnp.int32) for _ in range(NBUF)],
        [pltpu.VMEM((CH,), jnp.float32) for _ in range(NBUF)],
        [pltpu.VMEM((CH, H), jnp.float32) for _ in range(NBUF)],
        [pltpu.SemaphoreType.DMA for _ in range(NBUF)],
        [pltpu.SemaphoreType.DMA for _ in range(NBUF)],
        [pltpu.SemaphoreType.DMA for _ in range(NBUF)],
    ],
)
def _sc_scatter(y_hbm, gidx_hbm, cidx_hbm, cinv_hbm, out_hbm,
                agg_sh, gsb_v, csb_v, dc_vs, w_vs, rows_vs,
                gsems, wsems, ssems):
    c = lax.axis_index("c")
    s = lax.axis_index("s")
    wid = c * NS + s

    zero16 = jnp.zeros((L,), jnp.float32)
    rows0_v = rows_vs[0]

    def fill_z(i, carry):
        for j in range(H // L):
            rows0_v[i, pl.ds(j * L, L)] = zero16
        return carry
    lax.fori_loop(0, CH, fill_z, 0)

    zstart = pl.multiple_of(s * 624, 8)
    for i in range(624 // CH):
        pltpu.sync_copy(rows0_v, agg_sh.at[pl.ds(zstart + i * CH, CH)])
    if 624 % CH:
        pltpu.sync_copy(rows0_v.at[pl.ds(0, 624 % CH)],
                        agg_sh.at[pl.ds(zstart + (624 // CH) * CH, 624 % CH)])

    @pl.when(s == NS - 1)
    def _():
        pltpu.sync_copy(rows0_v.at[pl.ds(0, 16)], agg_sh.at[pl.ds(N - 16, 16)])

    plsc.subcore_barrier()

    base = wid * EPT

    def pre(k, b, wait_scatter):
        if wait_scatter:
            pltpu.make_async_copy(rows_vs[b], agg_sh.at[dc_vs[b]],
                                  ssems[b]).wait()
        for q in range(CH // L):
            ci = csb_v[pl.ds(k * CH + q * L, L)]
            dc_vs[b][pl.ds(q * L, L)] = ci & (NP - 1)
        gsl = gsb_v.at[pl.ds(k * CH, CH)]
        csl = csb_v.at[pl.ds(k * CH, CH)]
        pltpu.async_copy(y_hbm.at[gsl], rows_vs[b], gsems[b])
        pltpu.async_copy(cinv_hbm.at[csl], w_vs[b], wsems[b])

    def mid(k, b):
        gsl = gsb_v.at[pl.ds(k * CH, CH)]
        csl = csb_v.at[pl.ds(k * CH, CH)]
        pltpu.make_async_copy(y_hbm.at[gsl], rows_vs[b], gsems[b]).wait()
        pltpu.make_async_copy(cinv_hbm.at[csl], w_vs[b], wsems[b]).wait()
        w_v = w_vs[b]
        rows_v = rows_vs[b]

        def scale(r, carry2):
            r4 = r * 4
            ws = [plsc.load_gather(w_v, [jnp.full((L,), r4 + t, jnp.int32)])
                  for t in range(4)]
            for j in range(H // L):
                sl = pl.ds(j * L, L)
                for t in range(4):
                    rows_v[r4 + t, sl] = rows_v[r4 + t, sl] * ws[t]
            return carry2
        lax.fori_loop(0, CH // 4, scale, 0)
        pltpu.async_copy(rows_v, agg_sh.at[dc_vs[b]], ssems[b], add=True)

    for sb in range(NSB):
        off = pl.multiple_of(base + sb * SB, 8)
        pltpu.sync_copy(gidx_hbm.at[pl.ds(off, SB)], gsb_v)
        pltpu.sync_copy(cidx_hbm.at[pl.ds(off, SB)], csb_v)
        for b in range(NBUF):
            pre(b, b, wait_scatter=(sb > 0))

        def ring(i, carry):
            k = i * NBUF
            for b in range(NBUF):
                mid(k + b, b)
            for b in range(NBUF):
                pre(k + NBUF + b, b, wait_scatter=True)
            return carry
        lax.fori_loop(0, CPS // NBUF - 1, ring, 0)
        k0 = CPS - CPS % NBUF - NBUF
        for b in range(NBUF):
            mid(k0 + b, b)
        for k in range(CPS - CPS % NBUF, CPS):
            pre(k, k % NBUF, wait_scatter=True)
            mid(k, k % NBUF)
    for b in range(NBUF):
        pltpu.make_async_copy(rows_vs[b], agg_sh.at[dc_vs[b]], ssems[b]).wait()
    plsc.subcore_barrier()

    pltpu.sync_copy(agg_sh.at[pl.ds(zstart, 624)],
                    out_hbm.at[c, pl.ds(zstart, 624)])

    @pl.when(s == NS - 1)
    def _():
        pltpu.sync_copy(agg_sh.at[pl.ds(N - 16, 16)],
                        out_hbm.at[c, pl.ds(N - 16, 16)])


_PREC = lax.Precision.HIGHEST
BN = 1000
NB = N // BN
ER = E // 128


def _tc_cinv_body(cnt_ref, cinv_ref):
    cnt = cnt_ref[0] + cnt_ref[1]
    cinv_ref[...] = 1.0 / (3.0 * jnp.maximum(cnt, 1.0))


def _tc_cinv(cnt_part):
    c2 = cnt_part.reshape(NC, CROWS // 128, 128)
    out = pl.pallas_call(
        _tc_cinv_body,
        out_shape=jax.ShapeDtypeStruct((CROWS // 128, 128), jnp.float32),
    )(c2)
    return out.reshape(CROWS)


def _tc_prep_body(src_ref, dst_ref, et_ref, gidx_ref, cidx_ref):
    et = et_ref[...]
    gidx_ref[...] = et * N + src_ref[...]
    cidx_ref[...] = et * NP + dst_ref[...]


def _tc_prep(src, dst, et):
    gidx, cidx = pl.pallas_call(
        _tc_prep_body,
        out_shape=[
            jax.ShapeDtypeStruct((ER, 128), jnp.int32),
            jax.ShapeDtypeStruct((ER, 128), jnp.int32),
        ],
    )(src.reshape(ER, 128), dst.reshape(ER, 128), et.reshape(ER, 128))
    return gidx.reshape(E), cidx.reshape(E)


def _tc_dense1_body(x_ref, wn_ref, wr_ref, b_ref, y_ref, xr_ref):
    xb = x_ref[...]
    for e in range(NE):
        y_ref[e] = jnp.dot(xb, wn_ref[e], precision=_PREC)
    wrm = (wr_ref[0] + wr_ref[1] + wr_ref[2]) * (1.0 / 3.0)
    bm = (b_ref[0] + b_ref[1] + b_ref[2]) * (1.0 / 3.0)
    xr_ref[...] = jnp.dot(xb, wrm, precision=_PREC) + bm[None, :]


def _tc_dense1(x, Wn, Wr, b):
    return pl.pallas_call(
        _tc_dense1_body,
        grid=(NB,),
        in_specs=[
            pl.BlockSpec((BN, D), lambda i: (i, 0)),
            pl.BlockSpec((NE, D, H), lambda i: (0, 0, 0)),
            pl.BlockSpec((NE, D, H), lambda i: (0, 0, 0)),
            pl.BlockSpec((NE, H), lambda i: (0, 0)),
        ],
        out_specs=[
            pl.BlockSpec((NE, BN, H), lambda i: (0, i, 0)),
            pl.BlockSpec((BN, H), lambda i: (i, 0)),
        ],
        out_shape=[
            jax.ShapeDtypeStruct((NE, N, H), jnp.float32),
            jax.ShapeDtypeStruct((N, H), jnp.float32),
        ],
    )(x, Wn, Wr, b)


def _tc_dense2_body(xr_ref, agg_ref, wn_ref, wr_ref, b_ref, y_ref, xr2_ref):
    h = xr_ref[...] + agg_ref[0] + agg_ref[1]
    nrm = jnp.sqrt(jnp.sum(h * h, axis=-1, keepdims=True))
    h = h / jnp.maximum(nrm, 1e-12)
    h = jnp.maximum(h, 0.0)
    for e in range(NE):
        y_ref[e] = jnp.dot(h, wn_ref[e], precision=_PREC)
    wrm = (wr_ref[0] + wr_ref[1] + wr_ref[2]) * (1.0 / 3.0)
    bm = (b_ref[0] + b_ref[1] + b_ref[2]) * (1.0 / 3.0)
    xr2_ref[...] = jnp.dot(h, wrm, precision=_PREC) + bm[None, :]


def _tc_dense2(xr, agg, Wn, Wr, b):
    return pl.pallas_call(
        _tc_dense2_body,
        grid=(NB,),
        in_specs=[
            pl.BlockSpec((BN, H), lambda i: (i, 0)),
            pl.BlockSpec((NC, BN, H), lambda i: (0, i, 0)),
            pl.BlockSpec((NE, H, H), lambda i: (0, 0, 0)),
            pl.BlockSpec((NE, H, H), lambda i: (0, 0, 0)),
            pl.BlockSpec((NE, H), lambda i: (0, 0)),
        ],
        out_specs=[
            pl.BlockSpec((NE, BN, H), lambda i: (0, i, 0)),
            pl.BlockSpec((BN, H), lambda i: (i, 0)),
        ],
        out_shape=[
            jax.ShapeDtypeStruct((NE, N, H), jnp.float32),
            jax.ShapeDtypeStruct((N, H), jnp.float32),
        ],
    )(xr, agg, Wn, Wr, b)


def _tc_final_body(xr_ref, agg_ref, out_ref):
    out_ref[...] = xr_ref[...] + agg_ref[0] + agg_ref[1]


def _tc_final(xr, agg):
    return pl.pallas_call(
        _tc_final_body,
        grid=(NB,),
        in_specs=[
            pl.BlockSpec((BN, H), lambda i: (i, 0)),
            pl.BlockSpec((NC, BN, H), lambda i: (0, i, 0)),
        ],
        out_specs=pl.BlockSpec((BN, H), lambda i: (i, 0)),
        out_shape=jax.ShapeDtypeStruct((N, H), jnp.float32),
    )(xr, agg)


def kernel(x, edge_index, edge_type, beat_nodes, measure_nodes, beat_edges,
           measure_edges, W1r, W1n, b1, W2r, W2n, b2):
    del beat_nodes, measure_nodes, beat_edges, measure_edges
    src = edge_index[0]
    dst = edge_index[1]

    cnt_part = _sc_counts(edge_type, dst)
    cinv = _tc_cinv(cnt_part)

    gidx, cidx = _tc_prep(src, dst, edge_type)
    y1, xr1 = _tc_dense1(x, W1n, W1r, b1)
    agg1 = _sc_scatter(y1.reshape(NE * N, H), gidx, cidx, cinv)

    y2, xr2 = _tc_dense2(xr1, agg1, W2n, W2r, b2)
    agg2 = _sc_scatter(y2.reshape(NE * N, H), gidx, cidx, cinv)

    return _tc_final(xr2, agg2)
